# Initial kernel scaffold; baseline (speedup 1.0000x reference)
import jax
import jax.numpy as jnp
from jax.experimental import pallas as pl


def kernel(x, weight, ln_g, ln_b, e_W1, e_b1, e_W2, e_b2, n_W1, n_b1, n_W2, n_b2, o_W, o_b, a_W, a_b, edge_index):
    raise NotImplementedError("write your pallas kernel here")


# speedup vs baseline: 1.6654x; 1.6654x over previous
import functools

import jax
import jax.numpy as jnp
from jax import lax
from jax.experimental import pallas as pl
from jax.experimental.pallas import tpu as pltpu
from jax.experimental.pallas import tpu_sc as plsc

N = 10000
E = 320000
H = 128
R = 32
ED = 3 * H + R

NC = 2
NS = 16
NW = NC * NS
EPW = E // NW
CHUNK = 80
NCHUNKS = EPW // CHUNK
NPAD = 10240
RPT = NPAD // NS
RBLK = 128
CW = 16


def _swish(v):
    return v * (1.0 / (1.0 + jnp.exp(-v)))


def _prep_body(x_ref, g_ref, b_ref, w1a_ref, w1b_ref, xh_ref, p_ref, q_ref):
    x = x_ref[...]
    mu = jnp.mean(x, axis=1, keepdims=True)
    xc = x - mu
    var = jnp.mean(xc * xc, axis=1, keepdims=True)
    xh = xc * lax.rsqrt(var + 1e-5) * g_ref[...] + b_ref[...]
    xh_ref[...] = xh
    p_ref[...] = jnp.dot(xh, w1a_ref[...], preferred_element_type=jnp.float32)
    q_ref[...] = jnp.dot(xh, w1b_ref[...], preferred_element_type=jnp.float32)


def _prep(x, ln_g, ln_b, w1a, w1b):
    BN = 2000
    return pl.pallas_call(
        _prep_body,
        grid=(N // BN,),
        in_specs=[
            pl.BlockSpec((BN, H), lambda i: (i, 0)),
            pl.BlockSpec((1, H), lambda i: (0, 0)),
            pl.BlockSpec((1, H), lambda i: (0, 0)),
            pl.BlockSpec((H, H), lambda i: (0, 0)),
            pl.BlockSpec((H, H), lambda i: (0, 0)),
        ],
        out_specs=[pl.BlockSpec((BN, H), lambda i: (i, 0))] * 3,
        out_shape=[jax.ShapeDtypeStruct((N, H), jnp.float32)] * 3,
    )(x, ln_g.reshape(1, H), ln_b.reshape(1, H), w1a, w1b)


@functools.partial(
    pl.kernel,
    mesh=plsc.VectorSubcoreMesh(core_axis_name="c", subcore_axis_name="s"),
    out_type=[
        jax.ShapeDtypeStruct((E, H), jnp.float32),
        jax.ShapeDtypeStruct((E, H), jnp.float32),
    ],
    scratch_types=[
        pltpu.VMEM((CHUNK,), jnp.int32),
        pltpu.VMEM((CHUNK,), jnp.int32),
        pltpu.VMEM((CHUNK, H), jnp.float32),
        pltpu.VMEM((CHUNK, H), jnp.float32),
        pltpu.SemaphoreType.DMA,
        pltpu.SemaphoreType.DMA,
    ],
)
def _sc_gather(p_hbm, q_hbm, ii_hbm, jj_hbm, gp_hbm, gq_hbm,
               idx_i, idx_j, bufp, bufq, semp, semq):
    wid = lax.axis_index("s") * NC + lax.axis_index("c")
    base = wid * EPW

    def chunk(c, carry):
        off = base + c * CHUNK
        pltpu.sync_copy(ii_hbm.at[pl.ds(off, CHUNK)], idx_i)
        pltpu.sync_copy(jj_hbm.at[pl.ds(off, CHUNK)], idx_j)
        cp = pltpu.async_copy(p_hbm.at[idx_i], bufp, semp)
        cq = pltpu.async_copy(q_hbm.at[idx_j], bufq, semq)
        cp.wait()
        cq.wait()
        pltpu.sync_copy(bufp, gp_hbm.at[pl.ds(off, CHUNK)])
        pltpu.sync_copy(bufq, gq_hbm.at[pl.ds(off, CHUNK)])
        return carry

    lax.fori_loop(0, NCHUNKS, chunk, 0)


def _edge_body(w_ref, gp_ref, gq_ref, w1c_ref, b1_ref, w2_ref, b2_ref,
               arow_ref, ab_ref, ow_ref, ob_ref, mij_ref, eh_ref):
    w = w_ref[...]
    m = gp_ref[...] + gq_ref[...] + b1_ref[...]
    m = m + jnp.dot(w, w1c_ref[...], preferred_element_type=jnp.float32)
    m = _swish(m)
    m = _swish(jnp.dot(m, w2_ref[...], preferred_element_type=jnp.float32)
               + b2_ref[...])
    att = _swish(jnp.sum(m * arow_ref[...], axis=1, keepdims=True) + ab_ref[...])
    mij = m * att
    mij_ref[...] = mij
    eh_ref[...] = w + _swish(
        jnp.dot(mij, ow_ref[...], preferred_element_type=jnp.float32)
        + ob_ref[...])


def _edge(weight, gp, gq, w1c, e_b1, e_W2, e_b2, a_W, a_b, o_W, o_b):
    BE = 2000
    full = lambda shape: pl.BlockSpec(shape, lambda i: (0,) * len(shape))
    return pl.pallas_call(
        _edge_body,
        grid=(E // BE,),
        in_specs=[
            pl.BlockSpec((BE, ED), lambda i: (i, 0)),
            pl.BlockSpec((BE, H), lambda i: (i, 0)),
            pl.BlockSpec((BE, H), lambda i: (i, 0)),
            full((ED, H)),
            full((1, H)),
            full((H, H)),
            full((1, H)),
            full((1, H)),
            full((1, 1)),
            full((H, ED)),
            full((1, ED)),
        ],
        out_specs=[
            pl.BlockSpec((BE, H), lambda i: (i, 0)),
            pl.BlockSpec((BE, ED), lambda i: (i, 0)),
        ],
        out_shape=[
            jax.ShapeDtypeStruct((E, H), jnp.float32),
            jax.ShapeDtypeStruct((E, ED), jnp.float32),
        ],
    )(weight, gp, gq, w1c, e_b1.reshape(1, H), e_W2, e_b2.reshape(1, H),
      a_W.reshape(1, H), a_b.reshape(1, 1), o_W, o_b.reshape(1, ED))


EPT_SC = E // NS
NCHUNKS_SC = EPT_SC // CHUNK
NHALF = NPAD // 2
DSPREAD = 128
SROWS = NHALF + DSPREAD
DPT = NHALF // NS
NDB = DPT // CHUNK


@functools.partial(
    pl.kernel,
    mesh=plsc.VectorSubcoreMesh(core_axis_name="c", subcore_axis_name="s"),
    out_type=jax.ShapeDtypeStruct((NPAD, H), jnp.float32),
    scratch_types=[
        pltpu.VMEM((CHUNK,), jnp.int32),
        pltpu.VMEM((CHUNK, H), jnp.float32),
        pltpu.VMEM_SHARED((SROWS, H), jnp.float32),
    ],
)
def _sc_scatter(mij_hbm, ii_hbm, z_hbm, agg_hbm, idx_v, rows_v, agg_sh):
    cid = lax.axis_index("c")
    sid = lax.axis_index("s")

    pltpu.sync_copy(z_hbm, rows_v)
    dbase = sid * DPT
    for t in range(NDB):
        pltpu.sync_copy(rows_v, agg_sh.at[pl.ds(dbase + t * CHUNK, CHUNK)])
    plsc.subcore_barrier()

    base = sid * EPT_SC
    lo = cid * NHALF

    def chunk(c, carry):
        off = base + c * CHUNK
        pltpu.sync_copy(ii_hbm.at[pl.ds(off, CHUNK)], idx_v)
        pltpu.sync_copy(mij_hbm.at[pl.ds(off, CHUNK)], rows_v)
        for g in range(CHUNK // 16):
            v = idx_v[pl.ds(g * 16, 16)]
            local = v - lo
            inb = (local >= 0) & (local < NHALF)
            spread = (v & (DSPREAD - 1)) + NHALF
            idx_v[pl.ds(g * 16, 16)] = jnp.where(inb, local, spread)
        pltpu.sync_copy(rows_v, agg_sh.at[idx_v], add=True)
        return carry

    lax.fori_loop(0, NCHUNKS_SC, chunk, 0)
    plsc.subcore_barrier()

    for t in range(NDB):
        r0 = dbase + t * CHUNK
        pltpu.sync_copy(agg_sh.at[pl.ds(r0, CHUNK)], rows_v)
        pltpu.sync_copy(rows_v, agg_hbm.at[pl.ds(cid * NHALF + r0, CHUNK)])


@functools.partial(
    pl.kernel,
    mesh=plsc.VectorSubcoreMesh(core_axis_name="c", subcore_axis_name="s"),
    out_type=jax.ShapeDtypeStruct((NPAD, H), jnp.float32),
    scratch_types=[
        pltpu.VMEM((CHUNK,), jnp.int32),
        pltpu.VMEM((CHUNK, H), jnp.float32),
        pltpu.VMEM((CHUNK, H), jnp.float32),
        pltpu.VMEM_SHARED((SROWS, H), jnp.float32),
    ],
)
def _sc_count(ii_hbm, z_hbm, o_hbm, cnt_hbm, idx_v, ones_v, buf_v, cnt_sh):
    cid = lax.axis_index("c")
    sid = lax.axis_index("s")

    pltpu.sync_copy(z_hbm, buf_v)
    pltpu.sync_copy(o_hbm, ones_v)
    dbase = sid * DPT
    for t in range(NDB):
        pltpu.sync_copy(buf_v, cnt_sh.at[pl.ds(dbase + t * CHUNK, CHUNK)])
    plsc.subcore_barrier()

    base = sid * EPT_SC
    lo = cid * NHALF

    def chunk(c, carry):
        off = base + c * CHUNK
        pltpu.sync_copy(ii_hbm.at[pl.ds(off, CHUNK)], idx_v)
        for g in range(CHUNK // 16):
            v = idx_v[pl.ds(g * 16, 16)]
            local = v - lo
            inb = (local >= 0) & (local < NHALF)
            spread = (v & (DSPREAD - 1)) + NHALF
            idx_v[pl.ds(g * 16, 16)] = jnp.where(inb, local, spread)
        pltpu.sync_copy(ones_v, cnt_sh.at[idx_v], add=True)
        return carry

    lax.fori_loop(0, NCHUNKS_SC, chunk, 0)
    plsc.subcore_barrier()

    for t in range(NDB):
        r0 = dbase + t * CHUNK
        pltpu.sync_copy(cnt_sh.at[pl.ds(r0, CHUNK)], buf_v)
        pltpu.sync_copy(buf_v, cnt_hbm.at[pl.ds(cid * NHALF + r0, CHUNK)])


def _node_body(xh_ref, agg_ref, cnt_ref, w1a_ref, w1b_ref, b1_ref,
               w2_ref, b2_ref, out_ref):
    xh = xh_ref[...]
    agg = agg_ref[...] / jnp.maximum(cnt_ref[...], 1.0)
    h = _swish(jnp.dot(xh, w1a_ref[...], preferred_element_type=jnp.float32)
               + jnp.dot(agg, w1b_ref[...], preferred_element_type=jnp.float32)
               + b1_ref[...])
    h = _swish(jnp.dot(h, w2_ref[...], preferred_element_type=jnp.float32)
               + b2_ref[...])
    out_ref[...] = xh + h


def _node(xh, agg2, cnt2, w1a, w1b, n_b1, n_W2, n_b2):
    BN = 2000
    full = lambda shape: pl.BlockSpec(shape, lambda i: (0,) * len(shape))
    return pl.pallas_call(
        _node_body,
        grid=(N // BN,),
        in_specs=[
            pl.BlockSpec((BN, H), lambda i: (i, 0)),
            pl.BlockSpec((BN, H), lambda i: (i, 0)),
            pl.BlockSpec((BN, H), lambda i: (i, 0)),
            full((H, H)),
            full((H, H)),
            full((1, H)),
            full((H, H)),
            full((1, H)),
        ],
        out_specs=pl.BlockSpec((BN, H), lambda i: (i, 0)),
        out_shape=jax.ShapeDtypeStruct((N, H), jnp.float32),
    )(xh, agg2, cnt2, w1a, w1b, n_b1.reshape(1, H), n_W2, n_b2.reshape(1, H))


def kernel(x, weight, ln_g, ln_b, e_W1, e_b1, e_W2, e_b2,
           n_W1, n_b1, n_W2, n_b2, o_W, o_b, a_W, a_b, edge_index):
    ii = edge_index[0]
    jj = edge_index[1]
    w1a = e_W1[:H]
    w1b = e_W1[H:2 * H]
    w1c = e_W1[2 * H:]

    xh, p, q = _prep(x, ln_g, ln_b, w1a, w1b)
    gp, gq = _sc_gather(p, q, ii, jj)
    mij, eh = _edge(weight, gp, gq, w1c, e_b1, e_W2, e_b2, a_W, a_b, o_W, o_b)
    zrows = jnp.zeros((CHUNK, H), jnp.float32)
    cnt2 = _sc_count(ii, zrows, jnp.ones((CHUNK, H), jnp.float32))
    agg2 = _sc_scatter(mij, ii, zrows)
    xh_out = _node(xh, agg2, cnt2,
                   n_W1[:H], n_W1[H:], n_b1, n_W2, n_b2)
    return (xh_out, eh)


# speedup vs baseline: 1.6874x; 1.0132x over previous
import functools

import jax
import jax.numpy as jnp
from jax import lax
from jax.experimental import pallas as pl
from jax.experimental.pallas import tpu as pltpu
from jax.experimental.pallas import tpu_sc as plsc

N = 10000
E = 320000
H = 128
R = 32
ED = 3 * H + R

NC = 2
NS = 16
NW = NC * NS
EPW = E // NW
CHUNK = 80
NCHUNKS = EPW // CHUNK
NPAD = 10240
RPT = NPAD // NS
RBLK = 128
CW = 16


def _swish(v):
    return v * (1.0 / (1.0 + jnp.exp(-v)))


def _prep_body(x_ref, g_ref, b_ref, w1a_ref, w1b_ref, xh_ref, p_ref, q_ref):
    x = x_ref[...]
    mu = jnp.mean(x, axis=1, keepdims=True)
    xc = x - mu
    var = jnp.mean(xc * xc, axis=1, keepdims=True)
    xh = xc * lax.rsqrt(var + 1e-5) * g_ref[...] + b_ref[...]
    xh_ref[...] = xh
    p_ref[...] = jnp.dot(xh, w1a_ref[...], preferred_element_type=jnp.float32)
    q_ref[...] = jnp.dot(xh, w1b_ref[...], preferred_element_type=jnp.float32)


def _prep(x, ln_g, ln_b, w1a, w1b):
    BN = 2000
    return pl.pallas_call(
        _prep_body,
        grid=(N // BN,),
        in_specs=[
            pl.BlockSpec((BN, H), lambda i: (i, 0)),
            pl.BlockSpec((1, H), lambda i: (0, 0)),
            pl.BlockSpec((1, H), lambda i: (0, 0)),
            pl.BlockSpec((H, H), lambda i: (0, 0)),
            pl.BlockSpec((H, H), lambda i: (0, 0)),
        ],
        out_specs=[pl.BlockSpec((BN, H), lambda i: (i, 0))] * 3,
        out_shape=[jax.ShapeDtypeStruct((N, H), jnp.float32)] * 3,
    )(x, ln_g.reshape(1, H), ln_b.reshape(1, H), w1a, w1b)


GG = 5


@functools.partial(
    pl.kernel,
    mesh=plsc.VectorSubcoreMesh(core_axis_name="c", subcore_axis_name="s"),
    out_type=[
        jax.ShapeDtypeStruct((E, H), jnp.float32),
        jax.ShapeDtypeStruct((E, H), jnp.float32),
    ],
    scratch_types=(
        [pltpu.VMEM((CHUNK,), jnp.int32)] * (2 * GG)
        + [pltpu.VMEM((CHUNK, H), jnp.float32)] * (2 * GG)
        + [pltpu.SemaphoreType.DMA, pltpu.SemaphoreType.DMA]
    ),
)
def _sc_gather(p_hbm, q_hbm, ii_hbm, jj_hbm, gp_hbm, gq_hbm, *bufs):
    idx_i = bufs[0:GG]
    idx_j = bufs[GG:2 * GG]
    bufp = bufs[2 * GG:3 * GG]
    bufq = bufs[3 * GG:4 * GG]
    semg, semw = bufs[4 * GG], bufs[4 * GG + 1]
    wid = lax.axis_index("s") * NC + lax.axis_index("c")
    base = wid * EPW

    def outer(o, carry):
        gds = []
        for b in range(GG):
            off = base + (o * GG + b) * CHUNK
            pltpu.sync_copy(ii_hbm.at[pl.ds(off, CHUNK)], idx_i[b])
            pltpu.sync_copy(jj_hbm.at[pl.ds(off, CHUNK)], idx_j[b])
            gds.append(pltpu.async_copy(p_hbm.at[idx_i[b]], bufp[b], semg))
            gds.append(pltpu.async_copy(q_hbm.at[idx_j[b]], bufq[b], semg))
        wds = []
        for b in range(GG):
            off = base + (o * GG + b) * CHUNK
            gds[2 * b].wait()
            gds[2 * b + 1].wait()
            wds.append(pltpu.async_copy(bufp[b], gp_hbm.at[pl.ds(off, CHUNK)], semw))
            wds.append(pltpu.async_copy(bufq[b], gq_hbm.at[pl.ds(off, CHUNK)], semw))
        for d in wds:
            d.wait()
        return carry

    lax.fori_loop(0, NCHUNKS // GG, outer, 0)


def _edge_body(w_ref, gp_ref, gq_ref, w1c_ref, b1_ref, w2_ref, b2_ref,
               arow_ref, ab_ref, ow_ref, ob_ref, mij_ref, eh_ref):
    w = w_ref[...]
    m = gp_ref[...] + gq_ref[...] + b1_ref[...]
    m = m + jnp.dot(w, w1c_ref[...], preferred_element_type=jnp.float32)
    m = _swish(m)
    m = _swish(jnp.dot(m, w2_ref[...], preferred_element_type=jnp.float32)
               + b2_ref[...])
    att = _swish(jnp.sum(m * arow_ref[...], axis=1, keepdims=True) + ab_ref[...])
    mij = m * att
    mij_ref[...] = mij
    eh_ref[...] = w + _swish(
        jnp.dot(mij, ow_ref[...], preferred_element_type=jnp.float32)
        + ob_ref[...])


def _edge(weight, gp, gq, w1c, e_b1, e_W2, e_b2, a_W, a_b, o_W, o_b):
    BE = 2000
    full = lambda shape: pl.BlockSpec(shape, lambda i: (0,) * len(shape))
    return pl.pallas_call(
        _edge_body,
        grid=(E // BE,),
        in_specs=[
            pl.BlockSpec((BE, ED), lambda i: (i, 0)),
            pl.BlockSpec((BE, H), lambda i: (i, 0)),
            pl.BlockSpec((BE, H), lambda i: (i, 0)),
            full((ED, H)),
            full((1, H)),
            full((H, H)),
            full((1, H)),
            full((1, H)),
            full((1, 1)),
            full((H, ED)),
            full((1, ED)),
        ],
        out_specs=[
            pl.BlockSpec((BE, H), lambda i: (i, 0)),
            pl.BlockSpec((BE, ED), lambda i: (i, 0)),
        ],
        out_shape=[
            jax.ShapeDtypeStruct((E, H), jnp.float32),
            jax.ShapeDtypeStruct((E, ED), jnp.float32),
        ],
    )(weight, gp, gq, w1c, e_b1.reshape(1, H), e_W2, e_b2.reshape(1, H),
      a_W.reshape(1, H), a_b.reshape(1, 1), o_W, o_b.reshape(1, ED))


EPT_SC = E // NS
NCHUNKS_SC = EPT_SC // CHUNK
SG = 5
NHALF = NPAD // 2
DSPREAD = 128
SROWS = NHALF + DSPREAD
DPT = NHALF // NS
NDB = DPT // CHUNK


@functools.partial(
    pl.kernel,
    mesh=plsc.VectorSubcoreMesh(core_axis_name="c", subcore_axis_name="s"),
    out_type=jax.ShapeDtypeStruct((NPAD, H), jnp.float32),
    scratch_types=(
        [pltpu.VMEM((CHUNK,), jnp.int32)] * SG
        + [pltpu.VMEM((CHUNK, H), jnp.float32)] * SG
        + [pltpu.SemaphoreType.DMA,
           pltpu.VMEM_SHARED((SROWS, H), jnp.float32)]
    ),
)
def _sc_scatter(mij_hbm, ii_hbm, z_hbm, agg_hbm, *bufs):
    idx_v = bufs[0:SG]
    rows_v = bufs[SG:2 * SG]
    sem = bufs[2 * SG]
    agg_sh = bufs[2 * SG + 1]
    cid = lax.axis_index("c")
    sid = lax.axis_index("s")

    pltpu.sync_copy(z_hbm, rows_v[0])
    dbase = sid * DPT
    for t in range(NDB):
        pltpu.sync_copy(rows_v[0], agg_sh.at[pl.ds(dbase + t * CHUNK, CHUNK)])
    plsc.subcore_barrier()

    base = sid * EPT_SC
    lo = cid * NHALF

    def outer(o, carry):
        sds = []
        for b in range(SG):
            off = base + (o * SG + b) * CHUNK
            pltpu.sync_copy(ii_hbm.at[pl.ds(off, CHUNK)], idx_v[b])
            pltpu.sync_copy(mij_hbm.at[pl.ds(off, CHUNK)], rows_v[b])
            for g in range(CHUNK // 16):
                v = idx_v[b][pl.ds(g * 16, 16)]
                local = v - lo
                inb = (local >= 0) & (local < NHALF)
                spread = (v & (DSPREAD - 1)) + NHALF
                idx_v[b][pl.ds(g * 16, 16)] = jnp.where(inb, local, spread)
            sds.append(pltpu.async_copy(rows_v[b], agg_sh.at[idx_v[b]], sem, add=True))
        for d in sds:
            d.wait()
        return carry

    lax.fori_loop(0, NCHUNKS_SC // SG, outer, 0)
    plsc.subcore_barrier()

    for t in range(NDB):
        r0 = dbase + t * CHUNK
        pltpu.sync_copy(agg_sh.at[pl.ds(r0, CHUNK)], rows_v[0])
        pltpu.sync_copy(rows_v[0], agg_hbm.at[pl.ds(cid * NHALF + r0, CHUNK)])


@functools.partial(
    pl.kernel,
    mesh=plsc.VectorSubcoreMesh(core_axis_name="c", subcore_axis_name="s"),
    out_type=jax.ShapeDtypeStruct((NPAD, H), jnp.float32),
    scratch_types=(
        [pltpu.VMEM((CHUNK,), jnp.int32)] * SG
        + [pltpu.VMEM((CHUNK, H), jnp.float32),
           pltpu.VMEM((CHUNK, H), jnp.float32),
           pltpu.SemaphoreType.DMA,
           pltpu.VMEM_SHARED((SROWS, H), jnp.float32)]
    ),
)
def _sc_count(ii_hbm, z_hbm, o_hbm, cnt_hbm, *bufs):
    idx_v = bufs[0:SG]
    ones_v, buf_v, sem, cnt_sh = bufs[SG:SG + 4]
    cid = lax.axis_index("c")
    sid = lax.axis_index("s")

    pltpu.sync_copy(z_hbm, buf_v)
    pltpu.sync_copy(o_hbm, ones_v)
    dbase = sid * DPT
    for t in range(NDB):
        pltpu.sync_copy(buf_v, cnt_sh.at[pl.ds(dbase + t * CHUNK, CHUNK)])
    plsc.subcore_barrier()

    base = sid * EPT_SC
    lo = cid * NHALF

    def outer(o, carry):
        sds = []
        for b in range(SG):
            off = base + (o * SG + b) * CHUNK
            pltpu.sync_copy(ii_hbm.at[pl.ds(off, CHUNK)], idx_v[b])
            for g in range(CHUNK // 16):
                v = idx_v[b][pl.ds(g * 16, 16)]
                local = v - lo
                inb = (local >= 0) & (local < NHALF)
                spread = (v & (DSPREAD - 1)) + NHALF
                idx_v[b][pl.ds(g * 16, 16)] = jnp.where(inb, local, spread)
            sds.append(pltpu.async_copy(ones_v, cnt_sh.at[idx_v[b]], sem, add=True))
        for d in sds:
            d.wait()
        return carry

    lax.fori_loop(0, NCHUNKS_SC // SG, outer, 0)
    plsc.subcore_barrier()

    for t in range(NDB):
        r0 = dbase + t * CHUNK
        pltpu.sync_copy(cnt_sh.at[pl.ds(r0, CHUNK)], buf_v)
        pltpu.sync_copy(buf_v, cnt_hbm.at[pl.ds(cid * NHALF + r0, CHUNK)])


def _node_body(xh_ref, agg_ref, cnt_ref, w1a_ref, w1b_ref, b1_ref,
               w2_ref, b2_ref, out_ref):
    xh = xh_ref[...]
    agg = agg_ref[...] / jnp.maximum(cnt_ref[...], 1.0)
    h = _swish(jnp.dot(xh, w1a_ref[...], preferred_element_type=jnp.float32)
               + jnp.dot(agg, w1b_ref[...], preferred_element_type=jnp.float32)
               + b1_ref[...])
    h = _swish(jnp.dot(h, w2_ref[...], preferred_element_type=jnp.float32)
               + b2_ref[...])
    out_ref[...] = xh + h


def _node(xh, agg2, cnt2, w1a, w1b, n_b1, n_W2, n_b2):
    BN = 2000
    full = lambda shape: pl.BlockSpec(shape, lambda i: (0,) * len(shape))
    return pl.pallas_call(
        _node_body,
        grid=(N // BN,),
        in_specs=[
            pl.BlockSpec((BN, H), lambda i: (i, 0)),
            pl.BlockSpec((BN, H), lambda i: (i, 0)),
            pl.BlockSpec((BN, H), lambda i: (i, 0)),
            full((H, H)),
            full((H, H)),
            full((1, H)),
            full((H, H)),
            full((1, H)),
        ],
        out_specs=pl.BlockSpec((BN, H), lambda i: (i, 0)),
        out_shape=jax.ShapeDtypeStruct((N, H), jnp.float32),
    )(xh, agg2, cnt2, w1a, w1b, n_b1.reshape(1, H), n_W2, n_b2.reshape(1, H))


def kernel(x, weight, ln_g, ln_b, e_W1, e_b1, e_W2, e_b2,
           n_W1, n_b1, n_W2, n_b2, o_W, o_b, a_W, a_b, edge_index):
    ii = edge_index[0]
    jj = edge_index[1]
    w1a = e_W1[:H]
    w1b = e_W1[H:2 * H]
    w1c = e_W1[2 * H:]

    xh, p, q = _prep(x, ln_g, ln_b, w1a, w1b)
    gp, gq = _sc_gather(p, q, ii, jj)
    mij, eh = _edge(weight, gp, gq, w1c, e_b1, e_W2, e_b2, a_W, a_b, o_W, o_b)
    zrows = jnp.zeros((CHUNK, H), jnp.float32)
    cnt2 = _sc_count(ii, zrows, jnp.ones((CHUNK, H), jnp.float32))
    agg2 = _sc_scatter(mij, ii, zrows)
    xh_out = _node(xh, agg2, cnt2,
                   n_W1[:H], n_W1[H:], n_b1, n_W2, n_b2)
    return (xh_out, eh)


# speedup vs baseline: 1.6879x; 1.0003x over previous
import functools

import jax
import jax.numpy as jnp
from jax import lax
from jax.experimental import pallas as pl
from jax.experimental.pallas import tpu as pltpu
from jax.experimental.pallas import tpu_sc as plsc

N = 10000
E = 320000
H = 128
R = 32
ED = 3 * H + R

NC = 2
NS = 16
NW = NC * NS
EPW = E // NW
CHUNK = 80
NCHUNKS = EPW // CHUNK
NPAD = 10240
RPT = NPAD // NS
RBLK = 128
CW = 16


def _swish(v):
    return v * (1.0 / (1.0 + jnp.exp(-v)))


def _prep_body(x_ref, g_ref, b_ref, w1a_ref, w1b_ref, xh_ref, p_ref, q_ref):
    x = x_ref[...]
    mu = jnp.mean(x, axis=1, keepdims=True)
    xc = x - mu
    var = jnp.mean(xc * xc, axis=1, keepdims=True)
    xh = xc * lax.rsqrt(var + 1e-5) * g_ref[...] + b_ref[...]
    xh_ref[...] = xh
    p_ref[...] = jnp.dot(xh, w1a_ref[...], preferred_element_type=jnp.float32)
    q_ref[...] = jnp.dot(xh, w1b_ref[...], preferred_element_type=jnp.float32)


def _prep(x, ln_g, ln_b, w1a, w1b):
    BN = 2000
    return pl.pallas_call(
        _prep_body,
        grid=(N // BN,),
        in_specs=[
            pl.BlockSpec((BN, H), lambda i: (i, 0)),
            pl.BlockSpec((1, H), lambda i: (0, 0)),
            pl.BlockSpec((1, H), lambda i: (0, 0)),
            pl.BlockSpec((H, H), lambda i: (0, 0)),
            pl.BlockSpec((H, H), lambda i: (0, 0)),
        ],
        out_specs=[pl.BlockSpec((BN, H), lambda i: (i, 0))] * 3,
        out_shape=[jax.ShapeDtypeStruct((N, H), jnp.float32)] * 3,
    )(x, ln_g.reshape(1, H), ln_b.reshape(1, H), w1a, w1b)


GG = 5


@functools.partial(
    pl.kernel,
    mesh=plsc.VectorSubcoreMesh(core_axis_name="c", subcore_axis_name="s"),
    out_type=[
        jax.ShapeDtypeStruct((E, H), jnp.float32),
        jax.ShapeDtypeStruct((E, H), jnp.float32),
    ],
    scratch_types=(
        [pltpu.VMEM((CHUNK,), jnp.int32)] * (2 * GG)
        + [pltpu.VMEM((CHUNK, H), jnp.float32)] * (2 * GG)
        + [pltpu.SemaphoreType.DMA, pltpu.SemaphoreType.DMA]
    ),
)
def _sc_gather(p_hbm, q_hbm, ii_hbm, jj_hbm, gp_hbm, gq_hbm, *bufs):
    idx_i = bufs[0:GG]
    idx_j = bufs[GG:2 * GG]
    bufp = bufs[2 * GG:3 * GG]
    bufq = bufs[3 * GG:4 * GG]
    semg, semw = bufs[4 * GG], bufs[4 * GG + 1]
    wid = lax.axis_index("s") * NC + lax.axis_index("c")
    base = wid * EPW

    def outer(o, carry):
        gds = []
        for b in range(GG):
            off = base + (o * GG + b) * CHUNK
            pltpu.sync_copy(ii_hbm.at[pl.ds(off, CHUNK)], idx_i[b])
            pltpu.sync_copy(jj_hbm.at[pl.ds(off, CHUNK)], idx_j[b])
            gds.append(pltpu.async_copy(p_hbm.at[idx_i[b]], bufp[b], semg))
            gds.append(pltpu.async_copy(q_hbm.at[idx_j[b]], bufq[b], semg))
        wds = []
        for b in range(GG):
            off = base + (o * GG + b) * CHUNK
            gds[2 * b].wait()
            gds[2 * b + 1].wait()
            wds.append(pltpu.async_copy(bufp[b], gp_hbm.at[pl.ds(off, CHUNK)], semw))
            wds.append(pltpu.async_copy(bufq[b], gq_hbm.at[pl.ds(off, CHUNK)], semw))
        for d in wds:
            d.wait()
        return carry

    lax.fori_loop(0, NCHUNKS // GG, outer, 0)


def _edge_body(w_ref, gp_ref, gq_ref, w1c_ref, b1_ref, w2_ref, b2_ref,
               arow_ref, ab_ref, ow_ref, ob_ref, mij_ref, eh_ref):
    w = w_ref[...]
    m = gp_ref[...] + gq_ref[...] + b1_ref[...]
    m = m + jnp.dot(w, w1c_ref[...], preferred_element_type=jnp.float32)
    m = _swish(m)
    m = _swish(jnp.dot(m, w2_ref[...], preferred_element_type=jnp.float32)
               + b2_ref[...])
    att = _swish(jnp.sum(m * arow_ref[...], axis=1, keepdims=True) + ab_ref[...])
    mij = m * att
    mij_ref[...] = mij
    eh_ref[...] = w + _swish(
        jnp.dot(mij, ow_ref[...], preferred_element_type=jnp.float32)
        + ob_ref[...])


def _edge(weight, gp, gq, w1c, e_b1, e_W2, e_b2, a_W, a_b, o_W, o_b):
    BE = 2000
    full = lambda shape: pl.BlockSpec(shape, lambda i: (0,) * len(shape))
    return pl.pallas_call(
        _edge_body,
        grid=(E // BE,),
        in_specs=[
            pl.BlockSpec((BE, ED), lambda i: (i, 0)),
            pl.BlockSpec((BE, H), lambda i: (i, 0)),
            pl.BlockSpec((BE, H), lambda i: (i, 0)),
            full((ED, H)),
            full((1, H)),
            full((H, H)),
            full((1, H)),
            full((1, H)),
            full((1, 1)),
            full((H, ED)),
            full((1, ED)),
        ],
        out_specs=[
            pl.BlockSpec((BE, H), lambda i: (i, 0)),
            pl.BlockSpec((BE, ED), lambda i: (i, 0)),
        ],
        out_shape=[
            jax.ShapeDtypeStruct((E, H), jnp.float32),
            jax.ShapeDtypeStruct((E, ED), jnp.float32),
        ],
    )(weight, gp, gq, w1c, e_b1.reshape(1, H), e_W2, e_b2.reshape(1, H),
      a_W.reshape(1, H), a_b.reshape(1, 1), o_W, o_b.reshape(1, ED))


EPT_SC = E // NS
NCHUNKS_SC = EPT_SC // CHUNK
SG = 5
SGC = 10
NHALF = NPAD // 2
DSPREAD = 128
SROWS = NHALF + DSPREAD
DPT = NHALF // NS
NDB = DPT // CHUNK


@functools.partial(
    pl.kernel,
    mesh=plsc.VectorSubcoreMesh(core_axis_name="c", subcore_axis_name="s"),
    out_type=jax.ShapeDtypeStruct((NPAD, H), jnp.float32),
    scratch_types=(
        [pltpu.VMEM((CHUNK,), jnp.int32)] * SG
        + [pltpu.VMEM((CHUNK, H), jnp.float32)] * SG
        + [pltpu.SemaphoreType.DMA,
           pltpu.VMEM_SHARED((SROWS, H), jnp.float32)]
    ),
)
def _sc_scatter(mij_hbm, ii_hbm, z_hbm, agg_hbm, *bufs):
    idx_v = bufs[0:SG]
    rows_v = bufs[SG:2 * SG]
    sem = bufs[2 * SG]
    agg_sh = bufs[2 * SG + 1]
    cid = lax.axis_index("c")
    sid = lax.axis_index("s")

    pltpu.sync_copy(z_hbm, rows_v[0])
    dbase = sid * DPT
    for t in range(NDB):
        pltpu.sync_copy(rows_v[0], agg_sh.at[pl.ds(dbase + t * CHUNK, CHUNK)])
    plsc.subcore_barrier()

    base = sid * EPT_SC
    lo = cid * NHALF

    def outer(o, carry):
        sds = []
        for b in range(SG):
            off = base + (o * SG + b) * CHUNK
            pltpu.sync_copy(ii_hbm.at[pl.ds(off, CHUNK)], idx_v[b])
            pltpu.sync_copy(mij_hbm.at[pl.ds(off, CHUNK)], rows_v[b])
            for g in range(CHUNK // 16):
                v = idx_v[b][pl.ds(g * 16, 16)]
                local = v - lo
                inb = (local >= 0) & (local < NHALF)
                spread = (v & (DSPREAD - 1)) + NHALF
                idx_v[b][pl.ds(g * 16, 16)] = jnp.where(inb, local, spread)
            sds.append(pltpu.async_copy(rows_v[b], agg_sh.at[idx_v[b]], sem, add=True))
        for d in sds:
            d.wait()
        return carry

    lax.fori_loop(0, NCHUNKS_SC // SG, outer, 0)
    plsc.subcore_barrier()

    for t in range(NDB):
        r0 = dbase + t * CHUNK
        pltpu.sync_copy(agg_sh.at[pl.ds(r0, CHUNK)], rows_v[0])
        pltpu.sync_copy(rows_v[0], agg_hbm.at[pl.ds(cid * NHALF + r0, CHUNK)])


@functools.partial(
    pl.kernel,
    mesh=plsc.VectorSubcoreMesh(core_axis_name="c", subcore_axis_name="s"),
    out_type=jax.ShapeDtypeStruct((NPAD, H), jnp.float32),
    scratch_types=(
        [pltpu.VMEM((CHUNK,), jnp.int32)] * SGC
        + [pltpu.VMEM((CHUNK, H), jnp.float32),
           pltpu.VMEM((CHUNK, H), jnp.float32),
           pltpu.SemaphoreType.DMA,
           pltpu.VMEM_SHARED((SROWS, H), jnp.float32)]
    ),
)
def _sc_count(ii_hbm, z_hbm, o_hbm, cnt_hbm, *bufs):
    idx_v = bufs[0:SGC]
    ones_v, buf_v, sem, cnt_sh = bufs[SGC:SGC + 4]
    cid = lax.axis_index("c")
    sid = lax.axis_index("s")

    pltpu.sync_copy(z_hbm, buf_v)
    pltpu.sync_copy(o_hbm, ones_v)
    dbase = sid * DPT
    for t in range(NDB):
        pltpu.sync_copy(buf_v, cnt_sh.at[pl.ds(dbase + t * CHUNK, CHUNK)])
    plsc.subcore_barrier()

    base = sid * EPT_SC
    lo = cid * NHALF

    def outer(o, carry):
        sds = []
        for b in range(SGC):
            off = base + (o * SGC + b) * CHUNK
            pltpu.sync_copy(ii_hbm.at[pl.ds(off, CHUNK)], idx_v[b])
            for g in range(CHUNK // 16):
                v = idx_v[b][pl.ds(g * 16, 16)]
                local = v - lo
                inb = (local >= 0) & (local < NHALF)
                spread = (v & (DSPREAD - 1)) + NHALF
                idx_v[b][pl.ds(g * 16, 16)] = jnp.where(inb, local, spread)
            sds.append(pltpu.async_copy(ones_v, cnt_sh.at[idx_v[b]], sem, add=True))
        for d in sds:
            d.wait()
        return carry

    lax.fori_loop(0, NCHUNKS_SC // SGC, outer, 0)
    plsc.subcore_barrier()

    for t in range(NDB):
        r0 = dbase + t * CHUNK
        pltpu.sync_copy(cnt_sh.at[pl.ds(r0, CHUNK)], buf_v)
        pltpu.sync_copy(buf_v, cnt_hbm.at[pl.ds(cid * NHALF + r0, CHUNK)])


def _node_body(xh_ref, agg_ref, cnt_ref, w1a_ref, w1b_ref, b1_ref,
               w2_ref, b2_ref, out_ref):
    xh = xh_ref[...]
    agg = agg_ref[...] / jnp.maximum(cnt_ref[...], 1.0)
    h = _swish(jnp.dot(xh, w1a_ref[...], preferred_element_type=jnp.float32)
               + jnp.dot(agg, w1b_ref[...], preferred_element_type=jnp.float32)
               + b1_ref[...])
    h = _swish(jnp.dot(h, w2_ref[...], preferred_element_type=jnp.float32)
               + b2_ref[...])
    out_ref[...] = xh + h


def _node(xh, agg2, cnt2, w1a, w1b, n_b1, n_W2, n_b2):
    BN = 2000
    full = lambda shape: pl.BlockSpec(shape, lambda i: (0,) * len(shape))
    return pl.pallas_call(
        _node_body,
        grid=(N // BN,),
        in_specs=[
            pl.BlockSpec((BN, H), lambda i: (i, 0)),
            pl.BlockSpec((BN, H), lambda i: (i, 0)),
            pl.BlockSpec((BN, H), lambda i: (i, 0)),
            full((H, H)),
            full((H, H)),
            full((1, H)),
            full((H, H)),
            full((1, H)),
        ],
        out_specs=pl.BlockSpec((BN, H), lambda i: (i, 0)),
        out_shape=jax.ShapeDtypeStruct((N, H), jnp.float32),
    )(xh, agg2, cnt2, w1a, w1b, n_b1.reshape(1, H), n_W2, n_b2.reshape(1, H))


def kernel(x, weight, ln_g, ln_b, e_W1, e_b1, e_W2, e_b2,
           n_W1, n_b1, n_W2, n_b2, o_W, o_b, a_W, a_b, edge_index):
    ii = edge_index[0]
    jj = edge_index[1]
    w1a = e_W1[:H]
    w1b = e_W1[H:2 * H]
    w1c = e_W1[2 * H:]

    zrows = jnp.zeros((CHUNK, H), jnp.float32)
    cnt2 = _sc_count(ii, zrows, jnp.ones((CHUNK, H), jnp.float32))
    xh, p, q = _prep(x, ln_g, ln_b, w1a, w1b)
    gp, gq = _sc_gather(p, q, ii, jj)
    mij, eh = _edge(weight, gp, gq, w1c, e_b1, e_W2, e_b2, a_W, a_b, o_W, o_b)
    agg2 = _sc_scatter(mij, ii, zrows)
    xh_out = _node(xh, agg2, cnt2,
                   n_W1[:H], n_W1[H:], n_b1, n_W2, n_b2)
    return (xh_out, eh)


# speedup vs baseline: 1.7081x; 1.0119x over previous
import functools

import jax
import jax.numpy as jnp
from jax import lax
from jax.experimental import pallas as pl
from jax.experimental.pallas import tpu as pltpu
from jax.experimental.pallas import tpu_sc as plsc

N = 10000
E = 320000
H = 128
R = 32
ED = 3 * H + R

NC = 2
NS = 16
NW = NC * NS
EPW = E // NW
CHUNK = 80
NCHUNKS = EPW // CHUNK
NPAD = 10240
RPT = NPAD // NS
RBLK = 128
CW = 16


def _swish(v):
    return v * (1.0 / (1.0 + jnp.exp(-v)))


def _prep_body(x_ref, g_ref, b_ref, w1a_ref, w1b_ref, xh_ref, p_ref, q_ref):
    x = x_ref[...]
    mu = jnp.mean(x, axis=1, keepdims=True)
    xc = x - mu
    var = jnp.mean(xc * xc, axis=1, keepdims=True)
    xh = xc * lax.rsqrt(var + 1e-5) * g_ref[...] + b_ref[...]
    xh_ref[...] = xh
    p_ref[...] = jnp.dot(xh, w1a_ref[...], preferred_element_type=jnp.float32)
    q_ref[...] = jnp.dot(xh, w1b_ref[...], preferred_element_type=jnp.float32)


def _prep(x, ln_g, ln_b, w1a, w1b):
    BN = 2000
    return pl.pallas_call(
        _prep_body,
        grid=(N // BN,),
        in_specs=[
            pl.BlockSpec((BN, H), lambda i: (i, 0)),
            pl.BlockSpec((1, H), lambda i: (0, 0)),
            pl.BlockSpec((1, H), lambda i: (0, 0)),
            pl.BlockSpec((H, H), lambda i: (0, 0)),
            pl.BlockSpec((H, H), lambda i: (0, 0)),
        ],
        out_specs=[pl.BlockSpec((BN, H), lambda i: (i, 0))] * 3,
        out_shape=[jax.ShapeDtypeStruct((N, H), jnp.float32)] * 3,
    )(x, ln_g.reshape(1, H), ln_b.reshape(1, H), w1a, w1b)


GG = 5


@functools.partial(
    pl.kernel,
    mesh=plsc.VectorSubcoreMesh(core_axis_name="c", subcore_axis_name="s"),
    out_type=[
        jax.ShapeDtypeStruct((E, H), jnp.float32),
        jax.ShapeDtypeStruct((E, H), jnp.float32),
    ],
    scratch_types=(
        [pltpu.VMEM((CHUNK,), jnp.int32)] * (2 * GG)
        + [pltpu.VMEM((CHUNK, H), jnp.float32)] * (2 * GG)
        + [pltpu.SemaphoreType.DMA, pltpu.SemaphoreType.DMA]
    ),
)
def _sc_gather(p_hbm, q_hbm, ii_hbm, jj_hbm, gp_hbm, gq_hbm, *bufs):
    idx_i = bufs[0:GG]
    idx_j = bufs[GG:2 * GG]
    bufp = bufs[2 * GG:3 * GG]
    bufq = bufs[3 * GG:4 * GG]
    semg, semw = bufs[4 * GG], bufs[4 * GG + 1]
    wid = lax.axis_index("s") * NC + lax.axis_index("c")
    base = wid * EPW

    def outer(o, carry):
        gds = []
        for b in range(GG):
            off = base + (o * GG + b) * CHUNK
            pltpu.sync_copy(ii_hbm.at[pl.ds(off, CHUNK)], idx_i[b])
            pltpu.sync_copy(jj_hbm.at[pl.ds(off, CHUNK)], idx_j[b])
            gds.append(pltpu.async_copy(p_hbm.at[idx_i[b]], bufp[b], semg))
            gds.append(pltpu.async_copy(q_hbm.at[idx_j[b]], bufq[b], semg))
        wds = []
        for b in range(GG):
            off = base + (o * GG + b) * CHUNK
            gds[2 * b].wait()
            gds[2 * b + 1].wait()
            wds.append(pltpu.async_copy(bufp[b], gp_hbm.at[pl.ds(off, CHUNK)], semw))
            wds.append(pltpu.async_copy(bufq[b], gq_hbm.at[pl.ds(off, CHUNK)], semw))
        for d in wds:
            d.wait()
        return carry

    lax.fori_loop(0, NCHUNKS // GG, outer, 0)


def _edge_body(w_ref, gp_ref, gq_ref, w1c_ref, b1_ref, w2_ref, b2_ref,
               arow_ref, ab_ref, ow_ref, ob_ref, mij_ref, eh_ref):
    w = w_ref[...]
    m = gp_ref[...] + gq_ref[...] + b1_ref[...]
    m = m + jnp.dot(w, w1c_ref[...], preferred_element_type=jnp.float32)
    m = _swish(m)
    m = _swish(jnp.dot(m, w2_ref[...], preferred_element_type=jnp.float32)
               + b2_ref[...])
    att = _swish(jnp.sum(m * arow_ref[...], axis=1, keepdims=True) + ab_ref[...])
    mij = m * att
    mij_ref[...] = mij
    eh_ref[...] = w + _swish(
        jnp.dot(mij, ow_ref[...], preferred_element_type=jnp.float32)
        + ob_ref[...])


def _edge(weight, gp, gq, w1c, e_b1, e_W2, e_b2, a_W, a_b, o_W, o_b):
    BE = 4000
    full = lambda shape: pl.BlockSpec(shape, lambda i: (0,) * len(shape))
    return pl.pallas_call(
        _edge_body,
        grid=(E // BE,),
        in_specs=[
            pl.BlockSpec((BE, ED), lambda i: (i, 0)),
            pl.BlockSpec((BE, H), lambda i: (i, 0)),
            pl.BlockSpec((BE, H), lambda i: (i, 0)),
            full((ED, H)),
            full((1, H)),
            full((H, H)),
            full((1, H)),
            full((1, H)),
            full((1, 1)),
            full((H, ED)),
            full((1, ED)),
        ],
        out_specs=[
            pl.BlockSpec((BE, H), lambda i: (i, 0)),
            pl.BlockSpec((BE, ED), lambda i: (i, 0)),
        ],
        out_shape=[
            jax.ShapeDtypeStruct((E, H), jnp.float32),
            jax.ShapeDtypeStruct((E, ED), jnp.float32),
        ],
    )(weight, gp, gq, w1c, e_b1.reshape(1, H), e_W2, e_b2.reshape(1, H),
      a_W.reshape(1, H), a_b.reshape(1, 1), o_W, o_b.reshape(1, ED))


EPT_SC = E // NS
NCHUNKS_SC = EPT_SC // CHUNK
SG = 5
SGC = 10
NHALF = NPAD // 2
DSPREAD = 128
SROWS = NHALF + DSPREAD
DPT = NHALF // NS
NDB = DPT // CHUNK


@functools.partial(
    pl.kernel,
    mesh=plsc.VectorSubcoreMesh(core_axis_name="c", subcore_axis_name="s"),
    out_type=jax.ShapeDtypeStruct((NPAD, H), jnp.float32),
    scratch_types=(
        [pltpu.VMEM((CHUNK,), jnp.int32)] * SG
        + [pltpu.VMEM((CHUNK, H), jnp.float32)] * SG
        + [pltpu.SemaphoreType.DMA,
           pltpu.VMEM_SHARED((SROWS, H), jnp.float32)]
    ),
)
def _sc_scatter(mij_hbm, ii_hbm, z_hbm, agg_hbm, *bufs):
    idx_v = bufs[0:SG]
    rows_v = bufs[SG:2 * SG]
    sem = bufs[2 * SG]
    agg_sh = bufs[2 * SG + 1]
    cid = lax.axis_index("c")
    sid = lax.axis_index("s")

    pltpu.sync_copy(z_hbm, rows_v[0])
    dbase = sid * DPT
    for t in range(NDB):
        pltpu.sync_copy(rows_v[0], agg_sh.at[pl.ds(dbase + t * CHUNK, CHUNK)])
    plsc.subcore_barrier()

    base = sid * EPT_SC
    lo = cid * NHALF

    def outer(o, carry):
        sds = []
        for b in range(SG):
            off = base + (o * SG + b) * CHUNK
            pltpu.sync_copy(ii_hbm.at[pl.ds(off, CHUNK)], idx_v[b])
            pltpu.sync_copy(mij_hbm.at[pl.ds(off, CHUNK)], rows_v[b])
            for g in range(CHUNK // 16):
                v = idx_v[b][pl.ds(g * 16, 16)]
                local = v - lo
                inb = (local >= 0) & (local < NHALF)
                spread = (v & (DSPREAD - 1)) + NHALF
                idx_v[b][pl.ds(g * 16, 16)] = jnp.where(inb, local, spread)
            sds.append(pltpu.async_copy(rows_v[b], agg_sh.at[idx_v[b]], sem, add=True))
        for d in sds:
            d.wait()
        return carry

    lax.fori_loop(0, NCHUNKS_SC // SG, outer, 0)
    plsc.subcore_barrier()

    for t in range(NDB):
        r0 = dbase + t * CHUNK
        pltpu.sync_copy(agg_sh.at[pl.ds(r0, CHUNK)], rows_v[0])
        pltpu.sync_copy(rows_v[0], agg_hbm.at[pl.ds(cid * NHALF + r0, CHUNK)])


@functools.partial(
    pl.kernel,
    mesh=plsc.VectorSubcoreMesh(core_axis_name="c", subcore_axis_name="s"),
    out_type=jax.ShapeDtypeStruct((NPAD, H), jnp.float32),
    scratch_types=(
        [pltpu.VMEM((CHUNK,), jnp.int32)] * SGC
        + [pltpu.VMEM((CHUNK, H), jnp.float32),
           pltpu.VMEM((CHUNK, H), jnp.float32),
           pltpu.SemaphoreType.DMA,
           pltpu.VMEM_SHARED((SROWS, H), jnp.float32)]
    ),
)
def _sc_count(ii_hbm, z_hbm, o_hbm, cnt_hbm, *bufs):
    idx_v = bufs[0:SGC]
    ones_v, buf_v, sem, cnt_sh = bufs[SGC:SGC + 4]
    cid = lax.axis_index("c")
    sid = lax.axis_index("s")

    pltpu.sync_copy(z_hbm, buf_v)
    pltpu.sync_copy(o_hbm, ones_v)
    dbase = sid * DPT
    for t in range(NDB):
        pltpu.sync_copy(buf_v, cnt_sh.at[pl.ds(dbase + t * CHUNK, CHUNK)])
    plsc.subcore_barrier()

    base = sid * EPT_SC
    lo = cid * NHALF

    def outer(o, carry):
        sds = []
        for b in range(SGC):
            off = base + (o * SGC + b) * CHUNK
            pltpu.sync_copy(ii_hbm.at[pl.ds(off, CHUNK)], idx_v[b])
            for g in range(CHUNK // 16):
                v = idx_v[b][pl.ds(g * 16, 16)]
                local = v - lo
                inb = (local >= 0) & (local < NHALF)
                spread = (v & (DSPREAD - 1)) + NHALF
                idx_v[b][pl.ds(g * 16, 16)] = jnp.where(inb, local, spread)
            sds.append(pltpu.async_copy(ones_v, cnt_sh.at[idx_v[b]], sem, add=True))
        for d in sds:
            d.wait()
        return carry

    lax.fori_loop(0, NCHUNKS_SC // SGC, outer, 0)
    plsc.subcore_barrier()

    for t in range(NDB):
        r0 = dbase + t * CHUNK
        pltpu.sync_copy(cnt_sh.at[pl.ds(r0, CHUNK)], buf_v)
        pltpu.sync_copy(buf_v, cnt_hbm.at[pl.ds(cid * NHALF + r0, CHUNK)])


def _node_body(xh_ref, agg_ref, cnt_ref, w1a_ref, w1b_ref, b1_ref,
               w2_ref, b2_ref, out_ref):
    xh = xh_ref[...]
    agg = agg_ref[...] / jnp.maximum(cnt_ref[...], 1.0)
    h = _swish(jnp.dot(xh, w1a_ref[...], preferred_element_type=jnp.float32)
               + jnp.dot(agg, w1b_ref[...], preferred_element_type=jnp.float32)
               + b1_ref[...])
    h = _swish(jnp.dot(h, w2_ref[...], preferred_element_type=jnp.float32)
               + b2_ref[...])
    out_ref[...] = xh + h


def _node(xh, agg2, cnt2, w1a, w1b, n_b1, n_W2, n_b2):
    BN = 2000
    full = lambda shape: pl.BlockSpec(shape, lambda i: (0,) * len(shape))
    return pl.pallas_call(
        _node_body,
        grid=(N // BN,),
        in_specs=[
            pl.BlockSpec((BN, H), lambda i: (i, 0)),
            pl.BlockSpec((BN, H), lambda i: (i, 0)),
            pl.BlockSpec((BN, H), lambda i: (i, 0)),
            full((H, H)),
            full((H, H)),
            full((1, H)),
            full((H, H)),
            full((1, H)),
        ],
        out_specs=pl.BlockSpec((BN, H), lambda i: (i, 0)),
        out_shape=jax.ShapeDtypeStruct((N, H), jnp.float32),
    )(xh, agg2, cnt2, w1a, w1b, n_b1.reshape(1, H), n_W2, n_b2.reshape(1, H))


def kernel(x, weight, ln_g, ln_b, e_W1, e_b1, e_W2, e_b2,
           n_W1, n_b1, n_W2, n_b2, o_W, o_b, a_W, a_b, edge_index):
    ii = edge_index[0]
    jj = edge_index[1]
    w1a = e_W1[:H]
    w1b = e_W1[H:2 * H]
    w1c = e_W1[2 * H:]

    zrows = jnp.zeros((CHUNK, H), jnp.float32)
    cnt2 = _sc_count(ii, zrows, jnp.ones((CHUNK, H), jnp.float32))
    xh, p, q = _prep(x, ln_g, ln_b, w1a, w1b)
    gp, gq = _sc_gather(p, q, ii, jj)
    mij, eh = _edge(weight, gp, gq, w1c, e_b1, e_W2, e_b2, a_W, a_b, o_W, o_b)
    agg2 = _sc_scatter(mij, ii, zrows)
    xh_out = _node(xh, agg2, cnt2,
                   n_W1[:H], n_W1[H:], n_b1, n_W2, n_b2)
    return (xh_out, eh)


# speedup vs baseline: 1.7790x; 1.0416x over previous
import functools

import jax
import jax.numpy as jnp
from jax import lax
from jax.experimental import pallas as pl
from jax.experimental.pallas import tpu as pltpu
from jax.experimental.pallas import tpu_sc as plsc

N = 10000
E = 320000
H = 128
R = 32
ED = 3 * H + R

NC = 2
NS = 16
NW = NC * NS
EPW = E // NW
CHUNK = 80
NCHUNKS = EPW // CHUNK
NPAD = 10240
RPT = NPAD // NS
RBLK = 128
CW = 16


def _swish(v):
    return v * (1.0 / (1.0 + jnp.exp(-v)))


def _prep_body(x_ref, g_ref, b_ref, w1a_ref, w1b_ref, xh_ref, p_ref, q_ref):
    x = x_ref[...]
    mu = jnp.mean(x, axis=1, keepdims=True)
    xc = x - mu
    var = jnp.mean(xc * xc, axis=1, keepdims=True)
    xh = xc * lax.rsqrt(var + 1e-5) * g_ref[...] + b_ref[...]
    xh_ref[...] = xh
    p_ref[...] = jnp.dot(xh, w1a_ref[...], preferred_element_type=jnp.float32)
    q_ref[...] = jnp.dot(xh, w1b_ref[...], preferred_element_type=jnp.float32)


def _prep(x, ln_g, ln_b, w1a, w1b):
    BN = 2000
    return pl.pallas_call(
        _prep_body,
        grid=(N // BN,),
        in_specs=[
            pl.BlockSpec((BN, H), lambda i: (i, 0)),
            pl.BlockSpec((1, H), lambda i: (0, 0)),
            pl.BlockSpec((1, H), lambda i: (0, 0)),
            pl.BlockSpec((H, H), lambda i: (0, 0)),
            pl.BlockSpec((H, H), lambda i: (0, 0)),
        ],
        out_specs=[pl.BlockSpec((BN, H), lambda i: (i, 0))] * 3,
        out_shape=[jax.ShapeDtypeStruct((N, H), jnp.float32)] * 3,
    )(x, ln_g.reshape(1, H), ln_b.reshape(1, H), w1a, w1b)


GG = 5


@functools.partial(
    pl.kernel,
    mesh=plsc.VectorSubcoreMesh(core_axis_name="c", subcore_axis_name="s"),
    out_type=jax.ShapeDtypeStruct((E, H), jnp.float32),
    scratch_types=(
        [pltpu.VMEM((CHUNK,), jnp.int32)] * (2 * GG)
        + [pltpu.VMEM((CHUNK, H), jnp.float32)] * (2 * GG)
        + [pltpu.SemaphoreType.DMA, pltpu.SemaphoreType.DMA]
    ),
)
def _sc_gather(p_hbm, q_hbm, ii_hbm, jj_hbm, gs_hbm, *bufs):
    idx_i = bufs[0:GG]
    idx_j = bufs[GG:2 * GG]
    bufp = bufs[2 * GG:3 * GG]
    bufq = bufs[3 * GG:4 * GG]
    semg, semw = bufs[4 * GG], bufs[4 * GG + 1]
    wid = lax.axis_index("s") * NC + lax.axis_index("c")
    base = wid * EPW

    def outer(o, carry):
        gds = []
        for b in range(GG):
            off = base + (o * GG + b) * CHUNK
            pltpu.sync_copy(ii_hbm.at[pl.ds(off, CHUNK)], idx_i[b])
            pltpu.sync_copy(jj_hbm.at[pl.ds(off, CHUNK)], idx_j[b])
            gds.append(pltpu.async_copy(p_hbm.at[idx_i[b]], bufp[b], semg))
            gds.append(pltpu.async_copy(q_hbm.at[idx_j[b]], bufq[b], semg))
        wds = []
        for b in range(GG):
            off = base + (o * GG + b) * CHUNK
            gds[2 * b].wait()
            gds[2 * b + 1].wait()

            def vsum(r, carry2, _bp=bufp[b], _bq=bufq[b]):
                for k in range(H // 16):
                    _bp[r, pl.ds(k * 16, 16)] = (
                        _bp[r, pl.ds(k * 16, 16)] + _bq[r, pl.ds(k * 16, 16)])
                return carry2

            lax.fori_loop(0, CHUNK, vsum, 0)
            wds.append(pltpu.async_copy(bufp[b], gs_hbm.at[pl.ds(off, CHUNK)], semw))
        for d in wds:
            d.wait()
        return carry

    lax.fori_loop(0, NCHUNKS // GG, outer, 0)


def _edge_body(w_ref, gs_ref, w1c_ref, b1_ref, w2_ref, b2_ref,
               arow_ref, ab_ref, ow_ref, ob_ref, mij_ref, eh_ref):
    w = w_ref[...]
    m = gs_ref[...] + b1_ref[...]
    m = m + jnp.dot(w, w1c_ref[...], preferred_element_type=jnp.float32)
    m = _swish(m)
    m = _swish(jnp.dot(m, w2_ref[...], preferred_element_type=jnp.float32)
               + b2_ref[...])
    att = _swish(jnp.sum(m * arow_ref[...], axis=1, keepdims=True) + ab_ref[...])
    mij = m * att
    mij_ref[...] = mij
    eh_ref[...] = w + _swish(
        jnp.dot(mij, ow_ref[...], preferred_element_type=jnp.float32)
        + ob_ref[...])


def _edge(weight, gs, w1c, e_b1, e_W2, e_b2, a_W, a_b, o_W, o_b):
    BE = 4000
    full = lambda shape: pl.BlockSpec(shape, lambda i: (0,) * len(shape))
    return pl.pallas_call(
        _edge_body,
        grid=(E // BE,),
        in_specs=[
            pl.BlockSpec((BE, ED), lambda i: (i, 0)),
            pl.BlockSpec((BE, H), lambda i: (i, 0)),
            full((ED, H)),
            full((1, H)),
            full((H, H)),
            full((1, H)),
            full((1, H)),
            full((1, 1)),
            full((H, ED)),
            full((1, ED)),
        ],
        out_specs=[
            pl.BlockSpec((BE, H), lambda i: (i, 0)),
            pl.BlockSpec((BE, ED), lambda i: (i, 0)),
        ],
        out_shape=[
            jax.ShapeDtypeStruct((E, H), jnp.float32),
            jax.ShapeDtypeStruct((E, ED), jnp.float32),
        ],
    )(weight, gs, w1c, e_b1.reshape(1, H), e_W2, e_b2.reshape(1, H),
      a_W.reshape(1, H), a_b.reshape(1, 1), o_W, o_b.reshape(1, ED))


EPT_SC = E // NS
NCHUNKS_SC = EPT_SC // CHUNK
SG = 5
SGC = 10
NHALF = NPAD // 2
DSPREAD = 128
SROWS = NHALF + DSPREAD
DPT = NHALF // NS
NDB = DPT // CHUNK


@functools.partial(
    pl.kernel,
    mesh=plsc.VectorSubcoreMesh(core_axis_name="c", subcore_axis_name="s"),
    out_type=jax.ShapeDtypeStruct((NPAD, H), jnp.float32),
    scratch_types=(
        [pltpu.VMEM((CHUNK,), jnp.int32)] * SG
        + [pltpu.VMEM((CHUNK, H), jnp.float32)] * SG
        + [pltpu.SemaphoreType.DMA,
           pltpu.VMEM_SHARED((SROWS, H), jnp.float32)]
    ),
)
def _sc_scatter(mij_hbm, ii_hbm, z_hbm, agg_hbm, *bufs):
    idx_v = bufs[0:SG]
    rows_v = bufs[SG:2 * SG]
    sem = bufs[2 * SG]
    agg_sh = bufs[2 * SG + 1]
    cid = lax.axis_index("c")
    sid = lax.axis_index("s")

    pltpu.sync_copy(z_hbm, rows_v[0])
    dbase = sid * DPT
    for t in range(NDB):
        pltpu.sync_copy(rows_v[0], agg_sh.at[pl.ds(dbase + t * CHUNK, CHUNK)])
    plsc.subcore_barrier()

    base = sid * EPT_SC
    lo = cid * NHALF

    def outer(o, carry):
        sds = []
        for b in range(SG):
            off = base + (o * SG + b) * CHUNK
            pltpu.sync_copy(ii_hbm.at[pl.ds(off, CHUNK)], idx_v[b])
            pltpu.sync_copy(mij_hbm.at[pl.ds(off, CHUNK)], rows_v[b])
            for g in range(CHUNK // 16):
                v = idx_v[b][pl.ds(g * 16, 16)]
                local = v - lo
                inb = (local >= 0) & (local < NHALF)
                spread = (v & (DSPREAD - 1)) + NHALF
                idx_v[b][pl.ds(g * 16, 16)] = jnp.where(inb, local, spread)
            sds.append(pltpu.async_copy(rows_v[b], agg_sh.at[idx_v[b]], sem, add=True))
        for d in sds:
            d.wait()
        return carry

    lax.fori_loop(0, NCHUNKS_SC // SG, outer, 0)
    plsc.subcore_barrier()

    for t in range(NDB):
        r0 = dbase + t * CHUNK
        pltpu.sync_copy(agg_sh.at[pl.ds(r0, CHUNK)], rows_v[0])
        pltpu.sync_copy(rows_v[0], agg_hbm.at[pl.ds(cid * NHALF + r0, CHUNK)])


@functools.partial(
    pl.kernel,
    mesh=plsc.VectorSubcoreMesh(core_axis_name="c", subcore_axis_name="s"),
    out_type=jax.ShapeDtypeStruct((NPAD, H), jnp.float32),
    scratch_types=(
        [pltpu.VMEM((CHUNK,), jnp.int32)] * SGC
        + [pltpu.VMEM((CHUNK, H), jnp.float32),
           pltpu.VMEM((CHUNK, H), jnp.float32),
           pltpu.SemaphoreType.DMA,
           pltpu.VMEM_SHARED((SROWS, H), jnp.float32)]
    ),
)
def _sc_count(ii_hbm, z_hbm, o_hbm, cnt_hbm, *bufs):
    idx_v = bufs[0:SGC]
    ones_v, buf_v, sem, cnt_sh = bufs[SGC:SGC + 4]
    cid = lax.axis_index("c")
    sid = lax.axis_index("s")

    pltpu.sync_copy(z_hbm, buf_v)
    pltpu.sync_copy(o_hbm, ones_v)
    dbase = sid * DPT
    for t in range(NDB):
        pltpu.sync_copy(buf_v, cnt_sh.at[pl.ds(dbase + t * CHUNK, CHUNK)])
    plsc.subcore_barrier()

    base = sid * EPT_SC
    lo = cid * NHALF

    def outer(o, carry):
        sds = []
        for b in range(SGC):
            off = base + (o * SGC + b) * CHUNK
            pltpu.sync_copy(ii_hbm.at[pl.ds(off, CHUNK)], idx_v[b])
            for g in range(CHUNK // 16):
                v = idx_v[b][pl.ds(g * 16, 16)]
                local = v - lo
                inb = (local >= 0) & (local < NHALF)
                spread = (v & (DSPREAD - 1)) + NHALF
                idx_v[b][pl.ds(g * 16, 16)] = jnp.where(inb, local, spread)
            sds.append(pltpu.async_copy(ones_v, cnt_sh.at[idx_v[b]], sem, add=True))
        for d in sds:
            d.wait()
        return carry

    lax.fori_loop(0, NCHUNKS_SC // SGC, outer, 0)
    plsc.subcore_barrier()

    for t in range(NDB):
        r0 = dbase + t * CHUNK
        pltpu.sync_copy(cnt_sh.at[pl.ds(r0, CHUNK)], buf_v)
        pltpu.sync_copy(buf_v, cnt_hbm.at[pl.ds(cid * NHALF + r0, CHUNK)])


def _node_body(xh_ref, agg_ref, cnt_ref, w1a_ref, w1b_ref, b1_ref,
               w2_ref, b2_ref, out_ref):
    xh = xh_ref[...]
    agg = agg_ref[...] / jnp.maximum(cnt_ref[...], 1.0)
    h = _swish(jnp.dot(xh, w1a_ref[...], preferred_element_type=jnp.float32)
               + jnp.dot(agg, w1b_ref[...], preferred_element_type=jnp.float32)
               + b1_ref[...])
    h = _swish(jnp.dot(h, w2_ref[...], preferred_element_type=jnp.float32)
               + b2_ref[...])
    out_ref[...] = xh + h


def _node(xh, agg2, cnt2, w1a, w1b, n_b1, n_W2, n_b2):
    BN = 2000
    full = lambda shape: pl.BlockSpec(shape, lambda i: (0,) * len(shape))
    return pl.pallas_call(
        _node_body,
        grid=(N // BN,),
        in_specs=[
            pl.BlockSpec((BN, H), lambda i: (i, 0)),
            pl.BlockSpec((BN, H), lambda i: (i, 0)),
            pl.BlockSpec((BN, H), lambda i: (i, 0)),
            full((H, H)),
            full((H, H)),
            full((1, H)),
            full((H, H)),
            full((1, H)),
        ],
        out_specs=pl.BlockSpec((BN, H), lambda i: (i, 0)),
        out_shape=jax.ShapeDtypeStruct((N, H), jnp.float32),
    )(xh, agg2, cnt2, w1a, w1b, n_b1.reshape(1, H), n_W2, n_b2.reshape(1, H))


def kernel(x, weight, ln_g, ln_b, e_W1, e_b1, e_W2, e_b2,
           n_W1, n_b1, n_W2, n_b2, o_W, o_b, a_W, a_b, edge_index):
    ii = edge_index[0]
    jj = edge_index[1]
    w1a = e_W1[:H]
    w1b = e_W1[H:2 * H]
    w1c = e_W1[2 * H:]

    zrows = jnp.zeros((CHUNK, H), jnp.float32)
    cnt2 = _sc_count(ii, zrows, jnp.ones((CHUNK, H), jnp.float32))
    xh, p, q = _prep(x, ln_g, ln_b, w1a, w1b)
    gs = _sc_gather(p, q, ii, jj)
    mij, eh = _edge(weight, gs, w1c, e_b1, e_W2, e_b2, a_W, a_b, o_W, o_b)
    agg2 = _sc_scatter(mij, ii, zrows)
    xh_out = _node(xh, agg2, cnt2,
                   n_W1[:H], n_W1[H:], n_b1, n_W2, n_b2)
    return (xh_out, eh)
